# baseline (speedup 1.0000x reference)
import functools

import jax
import jax.numpy as jnp
from jax import lax
from jax.experimental import pallas as pl
from jax.experimental.pallas import tpu as pltpu
from jax.experimental.pallas import tpu_sc as plsc

B = 16384
XN = 26
H = 64
R = B * XN
NC, NS = 2, 16
NW = NC * NS
PW = R // NW
CHUNK = 128
CH = PW // CHUNK

_MESH = plsc.VectorSubcoreMesh(core_axis_name="c", subcore_axis_name="s")


@functools.partial(
    pl.kernel,
    out_type=jax.ShapeDtypeStruct((R, H), jnp.float32),
    mesh=_MESH,
    scratch_types=[
        pltpu.VMEM((CH, CHUNK), jnp.int32),
        pltpu.VMEM((CHUNK, H), jnp.float32),
        pltpu.SemaphoreType.DMA,
    ],
    compiler_params=pltpu.CompilerParams(use_tc_tiling_on_sc=False),
)
def _sc_gather(idx_hbm, table_hbm, out_hbm, idx_v, rows_v, sem):
    wid = lax.axis_index("s") * NC + lax.axis_index("c")
    base = wid * PW
    pltpu.sync_copy(idx_hbm.at[wid], idx_v)

    def step(j, carry):
        pltpu.async_copy(table_hbm.at[idx_v.at[j]], rows_v, sem).wait()
        pltpu.sync_copy(rows_v, out_hbm.at[pl.ds(base + j * CHUNK, CHUNK)])
        return carry

    lax.fori_loop(0, CH, step, 0)


def _tc_matmul_body(f_ref, w_ref, b_ref, o_ref):
    o_ref[...] = (
        jnp.dot(f_ref[...], w_ref[...], preferred_element_type=jnp.float32)
        + b_ref[...]
    )


def _tc_matmul(flat, Wt, b2):
    BM = 1024
    return pl.pallas_call(
        _tc_matmul_body,
        grid=(B // BM,),
        in_specs=[
            pl.BlockSpec((BM, XN * H), lambda i: (i, 0)),
            pl.BlockSpec((XN * H, H), lambda i: (0, 0)),
            pl.BlockSpec((1, H), lambda i: (0, 0)),
        ],
        out_specs=pl.BlockSpec((BM, H), lambda i: (i, 0)),
        out_shape=jax.ShapeDtypeStruct((B, H), jnp.float32),
    )(flat, Wt, b2)


def kernel(x, table, W, b):
    idx = x.astype(jnp.int32).reshape(NW, CH, CHUNK)
    emb = _sc_gather(idx, table)
    flat = emb.reshape(B, XN * H)
    return _tc_matmul(flat, W.T, b.reshape(1, H))


# speedup vs baseline: 1.1152x; 1.1152x over previous
import functools

import jax
import jax.numpy as jnp
from jax import lax
from jax.experimental import pallas as pl
from jax.experimental.pallas import tpu as pltpu
from jax.experimental.pallas import tpu_sc as plsc

B = 16384
XN = 26
H = 64
KT = XN // 2
R2 = KT * B
NC, NS = 2, 16
NW = NC * NS
PW = R2 // NW
CHUNK = 128
CH = PW // CHUNK

_MESH = plsc.VectorSubcoreMesh(core_axis_name="c", subcore_axis_name="s")


@functools.partial(
    pl.kernel,
    out_type=jax.ShapeDtypeStruct((R2, 2 * H), jnp.float32),
    mesh=_MESH,
    scratch_types=[
        pltpu.VMEM((CH, CHUNK), jnp.int32),
        pltpu.VMEM((CH, CHUNK), jnp.int32),
        pltpu.VMEM((CHUNK, H), jnp.float32),
        pltpu.VMEM((CHUNK, H), jnp.float32),
        pltpu.SemaphoreType.DMA,
    ],
    compiler_params=pltpu.CompilerParams(use_tc_tiling_on_sc=False),
)
def _sc_gather(idxe_hbm, idxo_hbm, table_hbm, out_hbm,
               idxe_v, idxo_v, bufe, bufo, sem):
    wid = lax.axis_index("s") * NC + lax.axis_index("c")
    base = wid * PW
    pltpu.sync_copy(idxe_hbm.at[wid], idxe_v)
    pltpu.sync_copy(idxo_hbm.at[wid], idxo_v)

    def step(j, carry):
        cpe = pltpu.async_copy(table_hbm.at[idxe_v.at[j]], bufe, sem)
        cpo = pltpu.async_copy(table_hbm.at[idxo_v.at[j]], bufo, sem)
        cpe.wait()
        cpo.wait()
        r0 = base + j * CHUNK
        pltpu.sync_copy(bufe, out_hbm.at[pl.ds(r0, CHUNK), pl.ds(0, H)])
        pltpu.sync_copy(bufo, out_hbm.at[pl.ds(r0, CHUNK), pl.ds(H, H)])
        return carry

    lax.fori_loop(0, CH, step, 0)


def _tc_matmul_body(e_ref, w_ref, b_ref, o_ref):
    t = pl.program_id(1)
    acc = jnp.dot(e_ref[...], w_ref[...], preferred_element_type=jnp.float32)

    @pl.when(t == 0)
    def _():
        o_ref[...] = acc + b_ref[...]

    @pl.when(t != 0)
    def _():
        o_ref[...] = o_ref[...] + acc


def _tc_matmul(emb2, Wt, b2):
    BM = 2048
    return pl.pallas_call(
        _tc_matmul_body,
        grid=(B // BM, KT),
        in_specs=[
            pl.BlockSpec((BM, 2 * H), lambda i, t: (t * (B // BM) + i, 0)),
            pl.BlockSpec((2 * H, H), lambda i, t: (t, 0)),
            pl.BlockSpec((1, H), lambda i, t: (0, 0)),
        ],
        out_specs=pl.BlockSpec((BM, H), lambda i, t: (i, 0)),
        out_shape=jax.ShapeDtypeStruct((B, H), jnp.float32),
    )(emb2, Wt, b2)


def kernel(x, table, W, b):
    xr = x.astype(jnp.int32).reshape(B, KT, 2)
    idxe = xr[:, :, 0].T.reshape(NW, CH, CHUNK)
    idxo = xr[:, :, 1].T.reshape(NW, CH, CHUNK)
    emb2 = _sc_gather(idxe, idxo, table)
    return _tc_matmul(emb2, W.T, b.reshape(1, H))
